# baseline (speedup 1.0000x reference)
import functools

import jax
import jax.numpy as jnp
from jax import lax
from jax.experimental import pallas as pl
from jax.experimental.pallas import tpu as pltpu
from jax.experimental.pallas import tpu_sc as plsc

_VOCAB = 1000000
_DIM = 64
_L = 16
_NW = 32

_NVT = _VOCAB // 128
_PER_W = 244
_TRM_ROWS = 7816

_mesh = plsc.VectorSubcoreMesh(core_axis_name="c", subcore_axis_name="s")


@functools.partial(
    pl.kernel,
    out_type=jax.ShapeDtypeStruct((_TRM_ROWS, 64, 128), jnp.float32),
    mesh=_mesh,
    scratch_types=[
        [pltpu.VMEM((64, 128), jnp.float32) for _ in range(2)],
        [pltpu.VMEM((64, 128), jnp.float32) for _ in range(2)],
        pltpu.VMEM((_L,), jnp.float32),
        [pltpu.SemaphoreType.DMA for _ in range(2)],
        [pltpu.SemaphoreType.DMA for _ in range(2)],
    ],
    compiler_params=pltpu.CompilerParams(use_tc_tiling_on_sc=True, needs_layout_passes=False),
)
def _k1(table_hbm, scale_hbm, trm_hbm, blk, tsp, scale_v, gsem, osem):
    wid = lax.axis_index("s") * 2 + lax.axis_index("c")
    pltpu.sync_copy(scale_hbm, scale_v)
    svec = scale_v[...]
    lo = wid * _PER_W
    iota = lax.iota(jnp.int32, _L)

    def in_copy(vt, b, w=128):
        return pltpu.make_async_copy(
            table_hbm.at[:, pl.ds(vt * 128, w)],
            blk[b].at[:, pl.ds(0, w)], gsem[b])

    def out_copy(vt, b):
        return pltpu.make_async_copy(tsp[b], trm_hbm.at[vt], osem[b])

    def transpose(b, ncol=128):
        def vloop(vc, _):
            cvec = jnp.zeros((_L,), jnp.int32) + vc
            r = vc >> 1
            c0 = (vc & 1) * 64
            for m in range(4):
                x = plsc.load_gather(blk[b], [iota + m * 16, cvec]) * svec
                tsp[b][r, pl.ds(c0 + m * 16, _L)] = x
            return 0

        lax.fori_loop(0, ncol, vloop, 0, unroll=2)

    in_copy(lo, 0).start()

    def body(g, _):
        for b in range(2):
            it = g * 2 + b
            vt = lo + it
            in_copy(vt, b).wait()

            @pl.when(it + 1 < _PER_W)
            def _():
                in_copy(vt + 1, 1 - b).start()

            @pl.when(it >= 2)
            def _():
                out_copy(vt - 2, b).wait()

            transpose(b)
            out_copy(vt, b).start()
        return 0

    lax.fori_loop(0, _PER_W // 2, body, 0)
    out_copy(lo + _PER_W - 2, 0).wait()
    out_copy(lo + _PER_W - 1, 1).wait()

    @pl.when(wid >= _NW - 4)
    def _():
        vt = _NW * _PER_W + (wid - (_NW - 4))
        in_copy(vt, 0).start()
        in_copy(vt, 0).wait()
        transpose(0)
        out_copy(vt, 0).start()
        out_copy(vt, 0).wait()

    @pl.when(wid == _NW - 5)
    def _():
        for d in range(_DIM):
            pltpu.async_copy(
                table_hbm.at[d, pl.ds(_NVT * 128, 64)],
                blk[1].at[d, pl.ds(0, 64)], gsem[1])
        for d in range(_DIM):
            pltpu.make_async_copy(
                table_hbm.at[d, pl.ds(_NVT * 128, 64)],
                blk[1].at[d, pl.ds(0, 64)], gsem[1]).wait()
        transpose(1, ncol=64)
        out_copy(_NVT, 1).start()
        out_copy(_NVT, 1).wait()


@functools.partial(
    pl.kernel,
    out_type=jax.ShapeDtypeStruct((50, 64, 16384), jnp.float32),
    mesh=_mesh,
    scratch_types=[
        [pltpu.VMEM((128,), jnp.int32) for _ in range(2)],
        [pltpu.VMEM((128,), jnp.int32) for _ in range(2)],
        [pltpu.VMEM((128,), jnp.int32) for _ in range(2)],
        [pltpu.VMEM((128, 128), jnp.float32) for _ in range(2)],
        [pltpu.VMEM((64, 128), jnp.float32) for _ in range(2)],
        [pltpu.SemaphoreType.DMA for _ in range(2)],
        [pltpu.SemaphoreType.DMA for _ in range(2)],
        [pltpu.SemaphoreType.DMA for _ in range(2)],
    ],
    compiler_params=pltpu.CompilerParams(use_tc_tiling_on_sc=True, needs_layout_passes=False),
)
def _k2(tok_hbm, trm_hbm, out_hbm, idxr, idx2, parv, rows, tsp, si, sg, so):
    wid = lax.axis_index("s") * 2 + lax.axis_index("c")
    base = wid * 200
    n = 200
    iota = lax.iota(jnp.int32, _L)

    def idx_copy(t, b):
        st = base + t
        return pltpu.make_async_copy(
            tok_hbm.at[st // 128, pl.ds((st % 128) * 128, 128)], idxr[b], si[b])

    def gat_copy(b):
        return pltpu.make_async_copy(trm_hbm.at[idx2[b]], rows[b], sg[b])

    def out_copy(t, b):
        st = base + t
        return pltpu.make_async_copy(
            tsp[b], out_hbm.at[st // 128, :, pl.ds((st % 128) * 128, 128)],
            so[b])

    def prep(b):
        for k in range(8):
            sl = pl.ds(k * 16, _L)
            tk = idxr[b][sl]
            idx2[b][sl] = tk >> 1
            parv[b][sl] = (tk & 1) << 6

    def transpose(b):
        for k in range(8):
            rowv = iota + k * 16
            park = parv[b][pl.ds(k * 16, _L)]

            def dloop(d, _):
                tsp[b][d, pl.ds(k * 16, _L)] = plsc.load_gather(
                    rows[b], [rowv, park + d])
                return 0

            lax.fori_loop(0, _DIM, dloop, 0, unroll=8)

    idx_copy(0, 0).start()
    idx_copy(1, 1).start()
    idx_copy(0, 0).wait()
    prep(0)
    gat_copy(0).start()

    def body(g, _):
        for b in range(2):
            t = g * 2 + b
            gat_copy(b).wait()

            @pl.when(t + 1 < n)
            def _():
                idx_copy(t + 1, 1 - b).wait()
                prep(1 - b)
                gat_copy(1 - b).start()

            @pl.when(t + 2 < n)
            def _():
                idx_copy(t + 2, b).start()

            @pl.when(t >= 2)
            def _():
                out_copy(t - 2, b).wait()

            transpose(b)
            out_copy(t, b).start()
        return 0

    lax.fori_loop(0, n // 2, body, 0)
    out_copy(n - 2, 0).wait()
    out_copy(n - 1, 1).wait()


def kernel(token_ids, embed_weight, scale):
    tok_t = token_ids.T
    if tok_t.dtype != jnp.int32:
        tok_t = tok_t.astype(jnp.int32)
    table_t = embed_weight.T
    scale_vec = jnp.broadcast_to(scale.astype(jnp.float32), (_L,))
    trm = _k1(table_t, scale_vec)
    trm_r = trm.reshape(_TRM_ROWS * 64, 128)
    out_t = _k2(tok_t, trm_r)
    return out_t.transpose(2, 0, 1)


# speedup vs baseline: 1.9634x; 1.9634x over previous
import functools

import jax
import jax.numpy as jnp
from jax import lax
from jax.experimental import pallas as pl
from jax.experimental.pallas import tpu as pltpu
from jax.experimental.pallas import tpu_sc as plsc

_VOCAB = 1000000
_DIM = 64
_L = 16
_NW = 32

_NVT = _VOCAB // 128
_PER_W = 244
_TRM_ROWS = 7816

_mesh = plsc.VectorSubcoreMesh(core_axis_name="c", subcore_axis_name="s")


@functools.partial(
    pl.kernel,
    out_type=jax.ShapeDtypeStruct((_TRM_ROWS, 64, 128), jnp.float32),
    mesh=_mesh,
    scratch_types=[
        [pltpu.VMEM((64, 128), jnp.float32) for _ in range(2)],
        [pltpu.VMEM((64, 128), jnp.float32) for _ in range(2)],
        pltpu.VMEM((_L,), jnp.float32),
        [pltpu.SemaphoreType.DMA for _ in range(2)],
        [pltpu.SemaphoreType.DMA for _ in range(2)],
    ],
    compiler_params=pltpu.CompilerParams(use_tc_tiling_on_sc=True, needs_layout_passes=False),
)
def _k1(table_hbm, scale_hbm, trm_hbm, blk, tsp, scale_v, gsem, osem):
    wid = lax.axis_index("s") * 2 + lax.axis_index("c")
    pltpu.sync_copy(scale_hbm, scale_v)
    svec = scale_v[...]
    lo = wid * _PER_W
    iota = lax.iota(jnp.int32, _L)

    def in_copy(vt, b, w=128):
        return pltpu.make_async_copy(
            table_hbm.at[:, pl.ds(vt * 128, w)],
            blk[b].at[:, pl.ds(0, w)], gsem[b])

    def out_copy(vt, b):
        return pltpu.make_async_copy(tsp[b], trm_hbm.at[vt], osem[b])

    def transpose(b, ncol=128):
        @plsc.parallel_loop(0, ncol, unroll=8)
        def _(vc):
            cvec = jnp.zeros((_L,), jnp.int32) + vc
            r = vc >> 1
            c0 = (vc & 1) * 64
            for m in range(4):
                x = plsc.load_gather(blk[b], [iota + m * 16, cvec]) * svec
                tsp[b][r, pl.ds(c0 + m * 16, _L)] = x

    in_copy(lo, 0).start()

    def body(g, _):
        for b in range(2):
            it = g * 2 + b
            vt = lo + it
            in_copy(vt, b).wait()

            @pl.when(it + 1 < _PER_W)
            def _():
                in_copy(vt + 1, 1 - b).start()

            @pl.when(it >= 2)
            def _():
                out_copy(vt - 2, b).wait()

            transpose(b)
            out_copy(vt, b).start()
        return 0

    lax.fori_loop(0, _PER_W // 2, body, 0)
    out_copy(lo + _PER_W - 2, 0).wait()
    out_copy(lo + _PER_W - 1, 1).wait()

    @pl.when(wid >= _NW - 4)
    def _():
        vt = _NW * _PER_W + (wid - (_NW - 4))
        in_copy(vt, 0).start()
        in_copy(vt, 0).wait()
        transpose(0)
        out_copy(vt, 0).start()
        out_copy(vt, 0).wait()

    @pl.when(wid == _NW - 5)
    def _():
        for d in range(_DIM):
            pltpu.async_copy(
                table_hbm.at[d, pl.ds(_NVT * 128, 64)],
                blk[1].at[d, pl.ds(0, 64)], gsem[1])
        for d in range(_DIM):
            pltpu.make_async_copy(
                table_hbm.at[d, pl.ds(_NVT * 128, 64)],
                blk[1].at[d, pl.ds(0, 64)], gsem[1]).wait()
        transpose(1, ncol=64)
        out_copy(_NVT, 1).start()
        out_copy(_NVT, 1).wait()


@functools.partial(
    pl.kernel,
    out_type=jax.ShapeDtypeStruct((50, 64, 16384), jnp.float32),
    mesh=_mesh,
    scratch_types=[
        [pltpu.VMEM((128,), jnp.int32) for _ in range(2)],
        [pltpu.VMEM((128,), jnp.int32) for _ in range(2)],
        [pltpu.VMEM((128,), jnp.int32) for _ in range(2)],
        [pltpu.VMEM((128, 128), jnp.float32) for _ in range(2)],
        [pltpu.VMEM((64, 128), jnp.float32) for _ in range(2)],
        [pltpu.SemaphoreType.DMA for _ in range(2)],
        [pltpu.SemaphoreType.DMA for _ in range(2)],
        [pltpu.SemaphoreType.DMA for _ in range(2)],
    ],
    compiler_params=pltpu.CompilerParams(use_tc_tiling_on_sc=True, needs_layout_passes=False),
)
def _k2(tok_hbm, trm_hbm, out_hbm, idxr, idx2, parv, rows, tsp, si, sg, so):
    wid = lax.axis_index("s") * 2 + lax.axis_index("c")
    base = wid * 200
    n = 200
    iota = lax.iota(jnp.int32, _L)

    def idx_copy(t, b):
        st = base + t
        return pltpu.make_async_copy(
            tok_hbm.at[st // 128, pl.ds((st % 128) * 128, 128)], idxr[b], si[b])

    def gat_copy(b):
        return pltpu.make_async_copy(trm_hbm.at[idx2[b]], rows[b], sg[b])

    def out_copy(t, b):
        st = base + t
        return pltpu.make_async_copy(
            tsp[b], out_hbm.at[st // 128, :, pl.ds((st % 128) * 128, 128)],
            so[b])

    def prep(b):
        for k in range(8):
            sl = pl.ds(k * 16, _L)
            tk = idxr[b][sl]
            idx2[b][sl] = tk >> 1
            parv[b][sl] = (tk & 1) << 6

    def transpose(b):
        for k in range(8):
            rowv = iota + k * 16
            park = parv[b][pl.ds(k * 16, _L)]

            @plsc.parallel_loop(0, _DIM, unroll=8)
            def _(d):
                tsp[b][d, pl.ds(k * 16, _L)] = plsc.load_gather(
                    rows[b], [rowv, park + d])

    idx_copy(0, 0).start()
    idx_copy(1, 1).start()
    idx_copy(0, 0).wait()
    prep(0)
    gat_copy(0).start()

    def body(g, _):
        for b in range(2):
            t = g * 2 + b
            gat_copy(b).wait()

            @pl.when(t + 1 < n)
            def _():
                idx_copy(t + 1, 1 - b).wait()
                prep(1 - b)
                gat_copy(1 - b).start()

            @pl.when(t + 2 < n)
            def _():
                idx_copy(t + 2, b).start()

            @pl.when(t >= 2)
            def _():
                out_copy(t - 2, b).wait()

            transpose(b)
            out_copy(t, b).start()
        return 0

    lax.fori_loop(0, n // 2, body, 0)
    out_copy(n - 2, 0).wait()
    out_copy(n - 1, 1).wait()


def kernel(token_ids, embed_weight, scale):
    tok_t = token_ids.T
    if tok_t.dtype != jnp.int32:
        tok_t = tok_t.astype(jnp.int32)
    table_t = embed_weight.T
    scale_vec = jnp.broadcast_to(scale.astype(jnp.float32), (_L,))
    trm = _k1(table_t, scale_vec)
    trm_r = trm.reshape(_TRM_ROWS * 64, 128)
    out_t = _k2(tok_t, trm_r)
    return out_t.transpose(2, 0, 1)


# speedup vs baseline: 4.2952x; 2.1877x over previous
import functools

import jax
import jax.numpy as jnp
from jax import lax
from jax.experimental import pallas as pl
from jax.experimental.pallas import tpu as pltpu
from jax.experimental.pallas import tpu_sc as plsc

_VOCAB = 1000000
_DIM = 64
_L = 16
_NW = 32

_NVT = _VOCAB // 128
_PER_W = 244
_TRM_ROWS = 7816

_mesh = plsc.VectorSubcoreMesh(core_axis_name="c", subcore_axis_name="s")


@functools.partial(
    pl.kernel,
    out_type=jax.ShapeDtypeStruct((_TRM_ROWS, 64, 128), jnp.float32),
    mesh=_mesh,
    scratch_types=[
        [pltpu.VMEM((64, 128), jnp.float32) for _ in range(2)],
        [pltpu.VMEM((64, 129), jnp.float32) for _ in range(2)],
        pltpu.VMEM((_L,), jnp.float32),
        [pltpu.SemaphoreType.DMA for _ in range(2)],
        [pltpu.SemaphoreType.DMA for _ in range(2)],
    ],
    compiler_params=pltpu.CompilerParams(use_tc_tiling_on_sc=True, needs_layout_passes=False),
)
def _k1(table_hbm, scale_hbm, trm_hbm, blk, tsp, scale_v, gsem, osem):
    wid = lax.axis_index("s") * 2 + lax.axis_index("c")
    pltpu.sync_copy(scale_hbm, scale_v)
    svec = scale_v[...]
    lo = wid * _PER_W
    iota = lax.iota(jnp.int32, _L)
    p8i = (iota & 1) << 3
    p64i = (iota & 1) << 6
    uvec = [m * 8 + (iota >> 1) for m in range(8)]

    def in_copy(vt, b, w=128):
        return pltpu.make_async_copy(
            table_hbm.at[:, pl.ds(vt * 128, w)],
            blk[b].at[:, pl.ds(0, w)], gsem[b])

    def out_copy(vt, b):
        return pltpu.make_async_copy(
            tsp[b].at[:, pl.ds(0, 128)], trm_hbm.at[vt], osem[b])

    def transpose(b, ncol=128):
        @plsc.parallel_loop(0, _DIM, unroll=4)
        def _(d):
            colv = ((d + p8i) & 63) + p64i
            for m in range(ncol // 16):
                x = blk[b][d, pl.ds(m * 16, _L)] * svec
                plsc.store_scatter(tsp[b], [uvec[m], colv], x)

    in_copy(lo, 0).start()

    def body(g, _):
        for b in range(2):
            it = g * 2 + b
            vt = lo + it
            in_copy(vt, b).wait()

            @pl.when(it + 1 < _PER_W)
            def _():
                in_copy(vt + 1, 1 - b).start()

            @pl.when(it >= 2)
            def _():
                out_copy(vt - 2, b).wait()

            transpose(b)
            out_copy(vt, b).start()
        return 0

    lax.fori_loop(0, _PER_W // 2, body, 0)
    out_copy(lo + _PER_W - 2, 0).wait()
    out_copy(lo + _PER_W - 1, 1).wait()

    @pl.when(wid >= _NW - 4)
    def _():
        vt = _NW * _PER_W + (wid - (_NW - 4))
        in_copy(vt, 0).start()
        in_copy(vt, 0).wait()
        transpose(0)
        out_copy(vt, 0).start()
        out_copy(vt, 0).wait()

    @pl.when(wid == _NW - 5)
    def _():
        for d in range(_DIM):
            pltpu.async_copy(
                table_hbm.at[d, pl.ds(_NVT * 128, 64)],
                blk[1].at[d, pl.ds(0, 64)], gsem[1])
        for d in range(_DIM):
            pltpu.make_async_copy(
                table_hbm.at[d, pl.ds(_NVT * 128, 64)],
                blk[1].at[d, pl.ds(0, 64)], gsem[1]).wait()
        transpose(1, ncol=64)
        out_copy(_NVT, 1).start()
        out_copy(_NVT, 1).wait()


@functools.partial(
    pl.kernel,
    out_type=jax.ShapeDtypeStruct((50, 64, 16384), jnp.float32),
    mesh=_mesh,
    scratch_types=[
        [pltpu.VMEM((128,), jnp.int32) for _ in range(2)],
        [pltpu.VMEM((128,), jnp.int32) for _ in range(2)],
        [pltpu.VMEM((128,), jnp.int32) for _ in range(2)],
        [pltpu.VMEM((128, 128), jnp.float32) for _ in range(2)],
        [pltpu.VMEM((64, 136), jnp.float32) for _ in range(2)],
        [pltpu.SemaphoreType.DMA for _ in range(2)],
        [pltpu.SemaphoreType.DMA for _ in range(2)],
        [pltpu.SemaphoreType.DMA for _ in range(2)],
    ],
    compiler_params=pltpu.CompilerParams(use_tc_tiling_on_sc=True, needs_layout_passes=False),
)
def _k2(tok_hbm, trm_hbm, out_hbm, idxr, idx2, parv, rows, tsp, si, sg, so):
    wid = lax.axis_index("s") * 2 + lax.axis_index("c")
    base = wid * 200
    n = 200
    iota = lax.iota(jnp.int32, _L)

    def idx_copy(t, b):
        st = base + t
        return pltpu.make_async_copy(
            tok_hbm.at[st // 128, pl.ds((st % 128) * 128, 128)], idxr[b], si[b])

    def gat_copy(b):
        return pltpu.make_async_copy(trm_hbm.at[idx2[b]], rows[b], sg[b])

    def out_copy(t, b):
        st = base + t
        return pltpu.make_async_copy(
            tsp[b].at[:, pl.ds(0, 128)],
            out_hbm.at[st // 128, :, pl.ds((st % 128) * 128, 128)],
            so[b])

    def prep(b):
        for k in range(8):
            sl = pl.ds(k * 16, _L)
            tk = idxr[b][sl]
            idx2[b][sl] = tk >> 1
            parv[b][sl] = (tk & 1) << 3

    def transpose(b):
        rowvs = [iota + k * 16 for k in range(8)]
        p8 = [parv[b][pl.ds(k * 16, _L)] for k in range(8)]
        p64 = [p << 3 for p in p8]

        @plsc.parallel_loop(0, _DIM, unroll=4)
        def _(d0):
            dvec = (d0 + iota) & 63
            for k in range(8):
                col = ((dvec + p8[k]) & 63) + p64[k]
                x = plsc.load_gather(rows[b], [rowvs[k], col])
                plsc.store_scatter(tsp[b], [dvec, rowvs[k]], x)

    idx_copy(0, 0).start()
    idx_copy(1, 1).start()
    idx_copy(0, 0).wait()
    prep(0)
    gat_copy(0).start()

    def body(g, _):
        for b in range(2):
            t = g * 2 + b
            gat_copy(b).wait()

            @pl.when(t + 1 < n)
            def _():
                idx_copy(t + 1, 1 - b).wait()
                prep(1 - b)
                gat_copy(1 - b).start()

            @pl.when(t + 2 < n)
            def _():
                idx_copy(t + 2, b).start()

            @pl.when(t >= 2)
            def _():
                out_copy(t - 2, b).wait()

            transpose(b)
            out_copy(t, b).start()
        return 0

    lax.fori_loop(0, n // 2, body, 0)
    out_copy(n - 2, 0).wait()
    out_copy(n - 1, 1).wait()


def kernel(token_ids, embed_weight, scale):
    tok_t = token_ids.T
    if tok_t.dtype != jnp.int32:
        tok_t = tok_t.astype(jnp.int32)
    table_t = embed_weight.T
    scale_vec = jnp.broadcast_to(scale.astype(jnp.float32), (_L,))
    trm = _k1(table_t, scale_vec)
    trm_r = trm.reshape(_TRM_ROWS * 64, 128)
    out_t = _k2(tok_t, trm_r)
    return out_t.transpose(2, 0, 1)


# speedup vs baseline: 4.9478x; 1.1519x over previous
import functools

import jax
import jax.numpy as jnp
from jax import lax
from jax.experimental import pallas as pl
from jax.experimental.pallas import tpu as pltpu
from jax.experimental.pallas import tpu_sc as plsc

_VOCAB = 1000000
_DIM = 64
_L = 16
_NW = 32

_NVT = _VOCAB // 128
_PER_W = 244
_TRM_ROWS = 7816

_mesh = plsc.VectorSubcoreMesh(core_axis_name="c", subcore_axis_name="s")


@functools.partial(
    pl.kernel,
    out_type=jax.ShapeDtypeStruct((_TRM_ROWS, 64, 128), jnp.float32),
    mesh=_mesh,
    scratch_types=[
        [pltpu.VMEM((64, 128), jnp.float32) for _ in range(4)],
        [pltpu.VMEM((64, 129), jnp.float32) for _ in range(2)],
        pltpu.VMEM((_L,), jnp.float32),
        [pltpu.SemaphoreType.DMA for _ in range(4)],
        [pltpu.SemaphoreType.DMA for _ in range(2)],
    ],
    compiler_params=pltpu.CompilerParams(use_tc_tiling_on_sc=True, needs_layout_passes=False),
)
def _k1(table_hbm, scale_hbm, trm_hbm, blk, tsp, scale_v, gsem, osem):
    wid = lax.axis_index("s") * 2 + lax.axis_index("c")
    pltpu.sync_copy(scale_hbm, scale_v)
    svec = scale_v[...]
    lo = wid * _PER_W
    iota = lax.iota(jnp.int32, _L)
    p8i = (iota & 1) << 3
    p64i = (iota & 1) << 6
    uvec = [m * 8 + (iota >> 1) for m in range(8)]

    def in_copy(vt, b, w=128):
        return pltpu.make_async_copy(
            table_hbm.at[:, pl.ds(vt * 128, w)],
            blk[b].at[:, pl.ds(0, w)], gsem[b])

    def out_copy(vt, b):
        return pltpu.make_async_copy(
            tsp[b].at[:, pl.ds(0, 128)], trm_hbm.at[vt], osem[b])

    def transpose(b, bt, ncol=128):
        @plsc.parallel_loop(0, _DIM, unroll=4)
        def _(d):
            colv = ((d + p8i) & 63) + p64i
            for m in range(ncol // 16):
                x = blk[b][d, pl.ds(m * 16, _L)] * svec
                plsc.store_scatter(tsp[bt], [uvec[m], colv], x)

    in_copy(lo, 0).start()
    in_copy(lo + 1, 1).start()

    def body(g, _):
        for b in range(4):
            it = g * 4 + b
            bt = b % 2
            vt = lo + it
            in_copy(vt, b).wait()

            @pl.when(it + 2 < _PER_W)
            def _():
                in_copy(vt + 2, (b + 2) % 4).start()

            @pl.when(it >= 2)
            def _():
                out_copy(vt - 2, bt).wait()

            transpose(b, bt)
            out_copy(vt, bt).start()
        return 0

    lax.fori_loop(0, _PER_W // 4, body, 0)
    out_copy(lo + _PER_W - 2, 0).wait()
    out_copy(lo + _PER_W - 1, 1).wait()

    @pl.when(wid >= _NW - 4)
    def _():
        vt = _NW * _PER_W + (wid - (_NW - 4))
        in_copy(vt, 0).start()
        in_copy(vt, 0).wait()
        transpose(0, 0)
        out_copy(vt, 0).start()
        out_copy(vt, 0).wait()

    @pl.when(wid == _NW - 5)
    def _():
        for d in range(_DIM):
            pltpu.async_copy(
                table_hbm.at[d, pl.ds(_NVT * 128, 64)],
                blk[1].at[d, pl.ds(0, 64)], gsem[1])
        for d in range(_DIM):
            pltpu.make_async_copy(
                table_hbm.at[d, pl.ds(_NVT * 128, 64)],
                blk[1].at[d, pl.ds(0, 64)], gsem[1]).wait()
        transpose(1, 1, ncol=64)
        out_copy(_NVT, 1).start()
        out_copy(_NVT, 1).wait()


@functools.partial(
    pl.kernel,
    out_type=jax.ShapeDtypeStruct((50, 64, 16384), jnp.float32),
    mesh=_mesh,
    scratch_types=[
        [pltpu.VMEM((128,), jnp.int32) for _ in range(4)],
        [pltpu.VMEM((128,), jnp.int32) for _ in range(4)],
        [pltpu.VMEM((128,), jnp.int32) for _ in range(4)],
        [pltpu.VMEM((128, 128), jnp.float32) for _ in range(4)],
        [pltpu.VMEM((64, 136), jnp.float32) for _ in range(2)],
        [pltpu.SemaphoreType.DMA for _ in range(4)],
        [pltpu.SemaphoreType.DMA for _ in range(4)],
        [pltpu.SemaphoreType.DMA for _ in range(2)],
    ],
    compiler_params=pltpu.CompilerParams(use_tc_tiling_on_sc=True, needs_layout_passes=False),
)
def _k2(tok_hbm, trm_hbm, out_hbm, idxr, idx2, parv, rows, tsp, si, sg, so):
    wid = lax.axis_index("s") * 2 + lax.axis_index("c")
    base = wid * 200
    n = 200
    iota = lax.iota(jnp.int32, _L)

    def idx_copy(t, b):
        st = base + t
        return pltpu.make_async_copy(
            tok_hbm.at[st // 128, pl.ds((st % 128) * 128, 128)], idxr[b], si[b])

    def gat_copy(b):
        return pltpu.make_async_copy(trm_hbm.at[idx2[b]], rows[b], sg[b])

    def out_copy(t, b):
        st = base + t
        return pltpu.make_async_copy(
            tsp[b].at[:, pl.ds(0, 128)],
            out_hbm.at[st // 128, :, pl.ds((st % 128) * 128, 128)],
            so[b])

    def prep(b):
        for k in range(8):
            sl = pl.ds(k * 16, _L)
            tk = idxr[b][sl]
            idx2[b][sl] = tk >> 1
            parv[b][sl] = (tk & 1) << 3

    def transpose(b, bt):
        rowvs = [iota + k * 16 for k in range(8)]
        p8 = [parv[b][pl.ds(k * 16, _L)] for k in range(8)]
        p64 = [p << 3 for p in p8]

        @plsc.parallel_loop(0, _DIM, unroll=4)
        def _(d0):
            dvec = (d0 + iota) & 63
            for k in range(8):
                col = ((dvec + p8[k]) & 63) + p64[k]
                x = plsc.load_gather(rows[b], [rowvs[k], col])
                plsc.store_scatter(tsp[bt], [dvec, rowvs[k]], x)

    idx_copy(0, 0).start()
    idx_copy(1, 1).start()
    idx_copy(0, 0).wait()
    prep(0)
    gat_copy(0).start()
    idx_copy(2, 2).start()
    idx_copy(1, 1).wait()
    prep(1)
    gat_copy(1).start()
    idx_copy(3, 3).start()

    def body(g, _):
        for b in range(4):
            t = g * 4 + b
            bt = b % 2
            gat_copy(b).wait()

            @pl.when(t + 2 < n)
            def _():
                b2 = (b + 2) % 4
                idx_copy(t + 2, b2).wait()
                prep(b2)
                gat_copy(b2).start()

            @pl.when(t + 4 < n)
            def _():
                idx_copy(t + 4, b).start()

            @pl.when(t >= 2)
            def _():
                out_copy(t - 2, bt).wait()

            transpose(b, bt)
            out_copy(t, bt).start()
        return 0

    lax.fori_loop(0, n // 4, body, 0)
    out_copy(n - 2, 0).wait()
    out_copy(n - 1, 1).wait()


def kernel(token_ids, embed_weight, scale):
    tok_t = token_ids.T
    if tok_t.dtype != jnp.int32:
        tok_t = tok_t.astype(jnp.int32)
    table_t = embed_weight.T
    scale_vec = jnp.broadcast_to(scale.astype(jnp.float32), (_L,))
    trm = _k1(table_t, scale_vec)
    trm_r = trm.reshape(_TRM_ROWS * 64, 128)
    out_t = _k2(tok_t, trm_r)
    return out_t.transpose(2, 0, 1)
